# baseline (speedup 1.0000x reference)
import functools

import jax
import jax.numpy as jnp
from jax.experimental import pallas as pl
from jax.experimental.pallas import tpu as pltpu
from jax.experimental.pallas import tpu_sc as plsc

_BETA = 0.25


def _e_prep_body(e_ref, em_ref, esq_ref):
    e = e_ref[...]
    em_ref[...] = e.astype(jnp.bfloat16)
    esq_ref[...] = jnp.sum(e * e, axis=1, keepdims=True)


def _e_prep(E, bk=1024):
    k_codes, d = E.shape
    return pl.pallas_call(
        _e_prep_body,
        grid=(k_codes // bk,),
        in_specs=[pl.BlockSpec((bk, d), lambda k: (k, 0))],
        out_specs=[
            pl.BlockSpec((bk, d), lambda k: (k, 0)),
            pl.BlockSpec((bk, 1), lambda k: (k, 0)),
        ],
        out_shape=[
            jax.ShapeDtypeStruct((k_codes, d), jnp.bfloat16),
            jax.ShapeDtypeStruct((k_codes, 1), jnp.float32),
        ],
    )(E)


def _dist_body(nb, kb, bn, bk, n_tokens, x_ref, em_ref, esq_ref,
               arg_ref, min_ref, loss_ref):
    n = pl.program_id(0)

    x = x_ref[...]
    x_sq = jnp.sum(x * x, axis=1, keepdims=True)
    xm2 = (-2.0 * x).astype(jnp.bfloat16)
    lanes = jax.lax.broadcasted_iota(jnp.int32, (bn, bk), 1).astype(
        jnp.float32)

    m_run = None
    a_run = None
    for c in range(kb):
        em = em_ref[pl.ds(c * bk, bk), :]
        s2 = jax.lax.dot_general(xm2, em, (((1,), (1,)), ((), ())),
                                 preferred_element_type=jnp.float32)
        e_sq = esq_ref[0:1, pl.ds(c * bk, bk)]
        dist = (x_sq + e_sq) + s2
        m = jnp.min(dist, axis=1, keepdims=True)
        masked = jnp.where(dist == m, lanes, jnp.float32(bk))
        a_loc = jnp.min(masked, axis=1, keepdims=True)
        a = a_loc.astype(jnp.int32) + c * bk
        if c == 0:
            m_run, a_run = m, a
        else:
            upd = m < m_run
            m_run = jnp.where(upd, m, m_run)
            a_run = jnp.where(upd, a, a_run)

    min_ref[...] = m_run
    arg_ref[...] = a_run

    part = jnp.sum(m_run, keepdims=True).reshape(1, 1)
    prev = jnp.where(n == 0, jnp.zeros((1, 1), jnp.float32), loss_ref[...])
    tot = prev + part
    loss_ref[...] = jnp.where(n == nb - 1, tot * (_BETA / n_tokens), tot)


def _argmin_min_loss(X, E_weight, bn=256, bk=1024, interpret=False):
    n_tokens, d = X.shape
    k_codes = E_weight.shape[0]
    nb, kb = n_tokens // bn, k_codes // bk
    em, esq_col = _e_prep(E_weight)
    esq = esq_col.reshape(1, k_codes)
    body = functools.partial(_dist_body, nb, kb, bn, bk, n_tokens)
    return pl.pallas_call(
        body,
        grid=(nb,),
        in_specs=[
            pl.BlockSpec((bn, d), lambda n: (n, 0)),
            pl.BlockSpec((k_codes, d), lambda n: (0, 0)),
            pl.BlockSpec((1, k_codes), lambda n: (0, 0)),
        ],
        out_specs=[
            pl.BlockSpec((bn, 1), lambda n: (n, 0)),
            pl.BlockSpec((bn, 1), lambda n: (n, 0)),
            pl.BlockSpec((1, 1), lambda n: (0, 0)),
        ],
        out_shape=[
            jax.ShapeDtypeStruct((n_tokens, 1), jnp.int32),
            jax.ShapeDtypeStruct((n_tokens, 1), jnp.float32),
            jax.ShapeDtypeStruct((1, 1), jnp.float32),
        ],
        compiler_params=pltpu.CompilerParams(
            dimension_semantics=("arbitrary",)),
        interpret=interpret,
    )(X, em, esq)


def _gather_rows(E_weight, argmins, window=128):
    n_tokens = argmins.shape[0]
    d = E_weight.shape[1]
    idx2 = argmins.reshape(1, n_tokens)
    mesh = plsc.VectorSubcoreMesh(core_axis_name="c", subcore_axis_name="s")

    @pl.kernel(out_type=jax.ShapeDtypeStruct((n_tokens, d), E_weight.dtype),
               mesh=mesh)
    def gather_kernel(e_hbm, i_hbm, o_hbm):
        def body(i_vmem, o_vmem):
            pltpu.sync_copy(e_hbm.at[i_vmem.at[0]], o_vmem)

        pltpu.emit_pipeline(
            body,
            grid=(n_tokens // window,),
            in_specs=[pl.BlockSpec((1, window), index_map=lambda i: (0, i))],
            out_specs=[pl.BlockSpec((window, d), index_map=lambda i: (i, 0))],
            core_axis_name=("c", "s"),
            dimension_semantics=(pltpu.PARALLEL,),
        )(i_hbm, o_hbm)

    return gather_kernel(E_weight, idx2)


def kernel(X, E_weight):
    n_tokens = X.shape[0]
    arg2, min2, loss2 = _argmin_min_loss(X, E_weight)
    argmins = arg2.reshape(n_tokens)
    min_dist = min2.reshape(n_tokens)
    loss = loss2[0, 0]
    z_st = _gather_rows(E_weight, argmins)
    return (z_st, loss, argmins, min_dist)


# speedup vs baseline: 1.0757x; 1.0757x over previous
import functools

import jax
import jax.numpy as jnp
from jax.experimental import pallas as pl
from jax.experimental.pallas import tpu as pltpu
from jax.experimental.pallas import tpu_sc as plsc

_BETA = 0.25


def _e_prep_body(e_ref, em_ref, esq_ref):
    e = e_ref[...]
    em_ref[...] = e.astype(jnp.bfloat16)
    esq_ref[...] = jnp.sum(e * e, axis=1, keepdims=True)


def _e_prep(E, bk=1024):
    k_codes, d = E.shape
    return pl.pallas_call(
        _e_prep_body,
        grid=(k_codes // bk,),
        in_specs=[pl.BlockSpec((bk, d), lambda k: (k, 0))],
        out_specs=[
            pl.BlockSpec((bk, d), lambda k: (k, 0)),
            pl.BlockSpec((bk, 1), lambda k: (k, 0)),
        ],
        out_shape=[
            jax.ShapeDtypeStruct((k_codes, d), jnp.bfloat16),
            jax.ShapeDtypeStruct((k_codes, 1), jnp.float32),
        ],
    )(E)


def _dist_body(nb, kb, bn, bk, n_tokens, x_ref, em_ref, esq_ref,
               arg_ref, min_ref, loss_ref):
    n = pl.program_id(0)

    x = x_ref[...]
    x_sq = jnp.sum(x * x, axis=1, keepdims=True)
    xm2 = (-2.0 * x).astype(jnp.bfloat16)
    lanes = jax.lax.broadcasted_iota(jnp.int32, (bn, bk), 1).astype(
        jnp.float32)

    m_run = None
    a_run = None
    for c in range(kb):
        em = em_ref[pl.ds(c * bk, bk), :]
        s2 = jax.lax.dot_general(xm2, em, (((1,), (1,)), ((), ())),
                                 preferred_element_type=jnp.float32)
        e_sq = esq_ref[0:1, pl.ds(c * bk, bk)]
        dist = e_sq + s2
        m = jnp.min(dist, axis=1, keepdims=True)
        masked = jnp.where(dist == m, lanes, jnp.float32(bk))
        a_loc = jnp.min(masked, axis=1, keepdims=True)
        a = a_loc.astype(jnp.int32) + c * bk
        if c == 0:
            m_run, a_run = m, a
        else:
            upd = m < m_run
            m_run = jnp.where(upd, m, m_run)
            a_run = jnp.where(upd, a, a_run)

    m_run = m_run + x_sq
    min_ref[...] = m_run
    arg_ref[...] = a_run

    part = jnp.sum(m_run, keepdims=True).reshape(1, 1)
    prev = jnp.where(n == 0, jnp.zeros((1, 1), jnp.float32), loss_ref[...])
    tot = prev + part
    loss_ref[...] = jnp.where(n == nb - 1, tot * (_BETA / n_tokens), tot)


def _argmin_min_loss(X, E_weight, bn=512, bk=1024, interpret=False):
    n_tokens, d = X.shape
    k_codes = E_weight.shape[0]
    nb, kb = n_tokens // bn, k_codes // bk
    em, esq_col = _e_prep(E_weight)
    esq = esq_col.reshape(1, k_codes)
    body = functools.partial(_dist_body, nb, kb, bn, bk, n_tokens)
    return pl.pallas_call(
        body,
        grid=(nb,),
        in_specs=[
            pl.BlockSpec((bn, d), lambda n: (n, 0)),
            pl.BlockSpec((k_codes, d), lambda n: (0, 0)),
            pl.BlockSpec((1, k_codes), lambda n: (0, 0)),
        ],
        out_specs=[
            pl.BlockSpec((bn, 1), lambda n: (n, 0)),
            pl.BlockSpec((bn, 1), lambda n: (n, 0)),
            pl.BlockSpec((1, 1), lambda n: (0, 0)),
        ],
        out_shape=[
            jax.ShapeDtypeStruct((n_tokens, 1), jnp.int32),
            jax.ShapeDtypeStruct((n_tokens, 1), jnp.float32),
            jax.ShapeDtypeStruct((1, 1), jnp.float32),
        ],
        compiler_params=pltpu.CompilerParams(
            dimension_semantics=("arbitrary",)),
        interpret=interpret,
    )(X, em, esq)


def _gather_rows(E_weight, argmins, window=128):
    n_tokens = argmins.shape[0]
    d = E_weight.shape[1]
    idx2 = argmins.reshape(1, n_tokens)
    mesh = plsc.VectorSubcoreMesh(core_axis_name="c", subcore_axis_name="s")

    @pl.kernel(out_type=jax.ShapeDtypeStruct((n_tokens, d), E_weight.dtype),
               mesh=mesh)
    def gather_kernel(e_hbm, i_hbm, o_hbm):
        def body(i_vmem, o_vmem):
            pltpu.sync_copy(e_hbm.at[i_vmem.at[0]], o_vmem)

        pltpu.emit_pipeline(
            body,
            grid=(n_tokens // window,),
            in_specs=[pl.BlockSpec((1, window), index_map=lambda i: (0, i))],
            out_specs=[pl.BlockSpec((window, d), index_map=lambda i: (i, 0))],
            core_axis_name=("c", "s"),
            dimension_semantics=(pltpu.PARALLEL,),
        )(i_hbm, o_hbm)

    return gather_kernel(E_weight, idx2)


def kernel(X, E_weight):
    n_tokens = X.shape[0]
    arg2, min2, loss2 = _argmin_min_loss(X, E_weight)
    argmins = arg2.reshape(n_tokens)
    min_dist = min2.reshape(n_tokens)
    loss = loss2[0, 0]
    z_st = _gather_rows(E_weight, argmins)
    return (z_st, loss, argmins, min_dist)


# speedup vs baseline: 1.2334x; 1.1466x over previous
import functools

import jax
import jax.numpy as jnp
from jax.experimental import pallas as pl
from jax.experimental.pallas import tpu as pltpu
from jax.experimental.pallas import tpu_sc as plsc

_BETA = 0.25


def _e_prep_body(e_ref, em_ref, esq_ref):
    e = e_ref[...]
    em_ref[...] = e.astype(jnp.bfloat16)
    esq_ref[...] = jnp.sum(e * e, axis=1, keepdims=True)


def _e_prep(E, bk=1024):
    k_codes, d = E.shape
    return pl.pallas_call(
        _e_prep_body,
        grid=(k_codes // bk,),
        in_specs=[pl.BlockSpec((bk, d), lambda k: (k, 0))],
        out_specs=[
            pl.BlockSpec((bk, d), lambda k: (k, 0)),
            pl.BlockSpec((bk, 1), lambda k: (k, 0)),
        ],
        out_shape=[
            jax.ShapeDtypeStruct((k_codes, d), jnp.bfloat16),
            jax.ShapeDtypeStruct((k_codes, 1), jnp.float32),
        ],
    )(E)


def _dist_body(nb, kb, bn, bk, n_tokens, x_ref, em_ref, esq_ref,
               arg_ref, min_ref, loss_ref):
    n = pl.program_id(0)

    x = x_ref[...]
    x_sq = jnp.sum(x * x, axis=1, keepdims=True)
    xm2 = (-2.0 * x).astype(jnp.bfloat16)
    lanef = jax.lax.broadcasted_iota(jnp.int32, (bn, 128), 1).astype(
        jnp.float32)

    m_run = None
    a_run = None
    for c in range(kb):
        em = em_ref[pl.ds(c * bk, bk), :]
        s2 = jax.lax.dot_general(xm2, em, (((1,), (1,)), ((), ())),
                                 preferred_element_type=jnp.float32)
        m128 = None
        c128 = None
        for j in range(bk // 128):
            e_sq = esq_ref[0:1, pl.ds(c * bk + j * 128, 128)]
            strip = e_sq + s2[:, j * 128:(j + 1) * 128]
            if j == 0:
                m128 = strip
                c128 = jnp.zeros((bn, 128), jnp.float32)
            else:
                lt = strip < m128
                m128 = jnp.where(lt, strip, m128)
                c128 = jnp.where(lt, jnp.float32(j), c128)
        m = jnp.min(m128, axis=1, keepdims=True)
        cand = jnp.where(m128 == m, c128 * jnp.float32(128.0) + lanef,
                         jnp.float32(bk))
        a_loc = jnp.min(cand, axis=1, keepdims=True)
        a = a_loc.astype(jnp.int32) + c * bk
        if c == 0:
            m_run, a_run = m, a
        else:
            upd = m < m_run
            m_run = jnp.where(upd, m, m_run)
            a_run = jnp.where(upd, a, a_run)

    m_run = m_run + x_sq
    min_ref[...] = m_run
    arg_ref[...] = a_run

    part = jnp.sum(m_run, keepdims=True).reshape(1, 1)
    prev = jnp.where(n == 0, jnp.zeros((1, 1), jnp.float32), loss_ref[...])
    tot = prev + part
    loss_ref[...] = jnp.where(n == nb - 1, tot * (_BETA / n_tokens), tot)


def _argmin_min_loss(X, E_weight, bn=512, bk=1024, interpret=False):
    n_tokens, d = X.shape
    k_codes = E_weight.shape[0]
    nb, kb = n_tokens // bn, k_codes // bk
    em, esq_col = _e_prep(E_weight)
    esq = esq_col.reshape(1, k_codes)
    body = functools.partial(_dist_body, nb, kb, bn, bk, n_tokens)
    return pl.pallas_call(
        body,
        grid=(nb,),
        in_specs=[
            pl.BlockSpec((bn, d), lambda n: (n, 0)),
            pl.BlockSpec((k_codes, d), lambda n: (0, 0)),
            pl.BlockSpec((1, k_codes), lambda n: (0, 0)),
        ],
        out_specs=[
            pl.BlockSpec((bn, 1), lambda n: (n, 0)),
            pl.BlockSpec((bn, 1), lambda n: (n, 0)),
            pl.BlockSpec((1, 1), lambda n: (0, 0)),
        ],
        out_shape=[
            jax.ShapeDtypeStruct((n_tokens, 1), jnp.int32),
            jax.ShapeDtypeStruct((n_tokens, 1), jnp.float32),
            jax.ShapeDtypeStruct((1, 1), jnp.float32),
        ],
        compiler_params=pltpu.CompilerParams(
            dimension_semantics=("arbitrary",)),
        interpret=interpret,
    )(X, em, esq)


def _gather_rows(E_weight, argmins, window=128):
    n_tokens = argmins.shape[0]
    d = E_weight.shape[1]
    idx2 = argmins.reshape(1, n_tokens)
    mesh = plsc.VectorSubcoreMesh(core_axis_name="c", subcore_axis_name="s")

    @pl.kernel(out_type=jax.ShapeDtypeStruct((n_tokens, d), E_weight.dtype),
               mesh=mesh)
    def gather_kernel(e_hbm, i_hbm, o_hbm):
        def body(i_vmem, o_vmem):
            pltpu.sync_copy(e_hbm.at[i_vmem.at[0]], o_vmem)

        pltpu.emit_pipeline(
            body,
            grid=(n_tokens // window,),
            in_specs=[pl.BlockSpec((1, window), index_map=lambda i: (0, i))],
            out_specs=[pl.BlockSpec((window, d), index_map=lambda i: (i, 0))],
            core_axis_name=("c", "s"),
            dimension_semantics=(pltpu.PARALLEL,),
        )(i_hbm, o_hbm)

    return gather_kernel(E_weight, idx2)


def kernel(X, E_weight):
    n_tokens = X.shape[0]
    arg2, min2, loss2 = _argmin_min_loss(X, E_weight)
    argmins = arg2.reshape(n_tokens)
    min_dist = min2.reshape(n_tokens)
    loss = loss2[0, 0]
    z_st = _gather_rows(E_weight, argmins)
    return (z_st, loss, argmins, min_dist)
